# baseline (speedup 1.0000x reference)
import jax
import jax.numpy as jnp
from jax.experimental import pallas as pl
from jax.experimental.pallas import tpu as pltpu

K = 5
BN = 512
INF = float("inf")


def _sym_kernel(src_ref, tgt_ref, rows_ref, cols_ref, colv_ref, colx_ref):
    i = pl.program_id(1)
    nb = pl.num_programs(1)
    src = src_ref[0]
    tgt = tgt_ref[0]

    cross = jnp.dot(src.astype(jnp.bfloat16), tgt.astype(jnp.bfloat16),
                    preferred_element_type=jnp.float32)
    sq0 = jnp.sum(src * src, axis=1, keepdims=True)
    sq1 = jnp.sum(tgt * tgt, axis=0, keepdims=True)
    d_sel = sq0 + sq1 - 2.0 * cross

    cross_hi = jnp.dot(src, tgt, preferred_element_type=jnp.float32,
                       precision=jax.lax.Precision.HIGHEST)
    d_exact = d_sel + 2.0 * (cross - cross_hi)

    vals = d_sel
    tops = []
    exacts = []
    for k in range(K):
        m = jnp.min(vals, axis=1, keepdims=True)
        mask = vals == m
        if k > 0:
            tops.append(m)
            exacts.append(jnp.sum(jnp.where(mask, d_exact, 0.0),
                                  axis=1, keepdims=True))
        vals = jnp.where(mask, INF, vals)

    w = [jnp.exp(t * -1000.0) for t in tops]
    s = w[0] + w[1] + w[2] + w[3] + 1e-5
    num = (w[0] * exacts[0] + w[1] * exacts[1]
           + w[2] * exacts[2] + w[3] * exacts[3])
    rows_ref[0] = 0.5 * num / s

    @pl.when(i == 0)
    def _init():
        colv_ref[...] = jnp.full_like(colv_ref, INF)
        colx_ref[...] = jnp.zeros_like(colx_ref)

    bvals = d_sel
    bv = []
    bx = []
    for k in range(K):
        m = jnp.min(bvals, axis=0, keepdims=True)
        mask = bvals == m
        bv.append(m)
        bx.append(jnp.sum(jnp.where(mask, d_exact, 0.0), axis=0, keepdims=True))
        bvals = jnp.where(mask, INF, bvals)

    cand_v = jnp.concatenate([colv_ref[:K]] + bv, axis=0)
    cand_x = jnp.concatenate([colx_ref[:K]] + bx, axis=0)
    cv = []
    cx = []
    for k in range(K):
        m = jnp.min(cand_v, axis=0, keepdims=True)
        mask = cand_v == m
        cv.append(m)
        cx.append(jnp.sum(jnp.where(mask, cand_x, 0.0), axis=0, keepdims=True))
        cand_v = jnp.where(mask, INF, cand_v)
    colv_ref[:K] = jnp.concatenate(cv, axis=0)
    colx_ref[:K] = jnp.concatenate(cx, axis=0)

    @pl.when(i == nb - 1)
    def _final():
        wc = [jnp.exp(cv[k] * -1000.0) for k in range(1, K)]
        sc = wc[0] + wc[1] + wc[2] + wc[3] + 1e-5
        numc = (wc[0] * cx[1] + wc[1] * cx[2]
                + wc[2] * cx[3] + wc[3] * cx[4])
        cols_ref[0] = 0.5 * numc / sc


def kernel(p0, p1):
    B, N, _ = p0.shape
    M = p1.shape[1]
    src_p = jnp.pad(p0, ((0, 0), (0, 0), (0, 5)))
    tgt_t = jnp.pad(p1, ((0, 0), (0, 0), (0, 5))).transpose(0, 2, 1)
    rows, cols = pl.pallas_call(
        _sym_kernel,
        grid=(B, N // BN),
        in_specs=[
            pl.BlockSpec((1, BN, 8), lambda b, i: (b, i, 0)),
            pl.BlockSpec((1, 8, M), lambda b, i: (b, 0, 0)),
        ],
        out_specs=[
            pl.BlockSpec((1, BN, 1), lambda b, i: (b, i, 0)),
            pl.BlockSpec((1, 1, M), lambda b, i: (b, 0, 0)),
        ],
        out_shape=[
            jax.ShapeDtypeStruct((B, N, 1), jnp.float32),
            jax.ShapeDtypeStruct((B, 1, M), jnp.float32),
        ],
        scratch_shapes=[
            pltpu.VMEM((K, M), jnp.float32),
            pltpu.VMEM((K, M), jnp.float32),
        ],
    )(src_p, tgt_t)
    return (jnp.sum(rows) + jnp.sum(cols)) / B


# speedup vs baseline: 1.0532x; 1.0532x over previous
import jax
import jax.numpy as jnp
from jax.experimental import pallas as pl
from jax.experimental.pallas import tpu as pltpu

K = 5
BN = 256
INF = float("inf")


def _sym_kernel(src_ref, tgt_ref, rows_ref, cols_ref, colv_ref, colx_ref):
    i = pl.program_id(1)
    nb = pl.num_programs(1)
    src = src_ref[0]
    tgt = tgt_ref[0]

    cross = jnp.dot(src.astype(jnp.bfloat16), tgt.astype(jnp.bfloat16),
                    preferred_element_type=jnp.float32)
    sq0 = jnp.sum(src * src, axis=1, keepdims=True)
    sq1 = jnp.sum(tgt * tgt, axis=0, keepdims=True)
    d_sel = sq0 + sq1 - 2.0 * cross

    d_exact = jnp.zeros_like(d_sel)
    for c in range(3):
        diff = src[:, c:c + 1] - tgt[c:c + 1, :]
        d_exact = d_exact + diff * diff

    vals = d_sel
    tops = []
    exacts = []
    for k in range(K):
        m = jnp.min(vals, axis=1, keepdims=True)
        mask = vals == m
        if k > 0:
            tops.append(m)
            exacts.append(jnp.sum(jnp.where(mask, d_exact, 0.0),
                                  axis=1, keepdims=True))
        vals = jnp.where(mask, INF, vals)

    w = [jnp.exp(t * -1000.0) for t in tops]
    s = w[0] + w[1] + w[2] + w[3] + 1e-5
    num = (w[0] * exacts[0] + w[1] * exacts[1]
           + w[2] * exacts[2] + w[3] * exacts[3])
    rows_ref[0] = 0.5 * num / s

    @pl.when(i == 0)
    def _init():
        colv_ref[...] = jnp.full_like(colv_ref, INF)
        colx_ref[...] = jnp.zeros_like(colx_ref)

    bvals = d_sel
    bv = []
    bx = []
    for k in range(K):
        m = jnp.min(bvals, axis=0, keepdims=True)
        mask = bvals == m
        bv.append(m)
        bx.append(jnp.sum(jnp.where(mask, d_exact, 0.0), axis=0, keepdims=True))
        bvals = jnp.where(mask, INF, bvals)

    cand_v = jnp.concatenate([colv_ref[:K]] + bv, axis=0)
    cand_x = jnp.concatenate([colx_ref[:K]] + bx, axis=0)
    cv = []
    cx = []
    for k in range(K):
        m = jnp.min(cand_v, axis=0, keepdims=True)
        mask = cand_v == m
        cv.append(m)
        cx.append(jnp.sum(jnp.where(mask, cand_x, 0.0), axis=0, keepdims=True))
        cand_v = jnp.where(mask, INF, cand_v)
    colv_ref[:K] = jnp.concatenate(cv, axis=0)
    colx_ref[:K] = jnp.concatenate(cx, axis=0)

    @pl.when(i == nb - 1)
    def _final():
        wc = [jnp.exp(cv[k] * -1000.0) for k in range(1, K)]
        sc = wc[0] + wc[1] + wc[2] + wc[3] + 1e-5
        numc = (wc[0] * cx[1] + wc[1] * cx[2]
                + wc[2] * cx[3] + wc[3] * cx[4])
        cols_ref[0] = 0.5 * numc / sc


def kernel(p0, p1):
    B, N, _ = p0.shape
    M = p1.shape[1]
    src_p = jnp.pad(p0, ((0, 0), (0, 0), (0, 5)))
    tgt_t = jnp.pad(p1, ((0, 0), (0, 0), (0, 5))).transpose(0, 2, 1)
    rows, cols = pl.pallas_call(
        _sym_kernel,
        grid=(B, N // BN),
        in_specs=[
            pl.BlockSpec((1, BN, 8), lambda b, i: (b, i, 0)),
            pl.BlockSpec((1, 8, M), lambda b, i: (b, 0, 0)),
        ],
        out_specs=[
            pl.BlockSpec((1, BN, 1), lambda b, i: (b, i, 0)),
            pl.BlockSpec((1, 1, M), lambda b, i: (b, 0, 0)),
        ],
        out_shape=[
            jax.ShapeDtypeStruct((B, N, 1), jnp.float32),
            jax.ShapeDtypeStruct((B, 1, M), jnp.float32),
        ],
        scratch_shapes=[
            pltpu.VMEM((K, M), jnp.float32),
            pltpu.VMEM((K, M), jnp.float32),
        ],
    )(src_p, tgt_t)
    return (jnp.sum(rows) + jnp.sum(cols)) / B


# speedup vs baseline: 1.1561x; 1.0976x over previous
import jax
import jax.numpy as jnp
from jax.experimental import pallas as pl
from jax.experimental.pallas import tpu as pltpu

K = 5
BN = 1024
INF = float("inf")


def _sym_kernel(src_ref, tgt_ref, rows_ref, cols_ref, colv_ref, colx_ref):
    i = pl.program_id(1)
    nb = pl.num_programs(1)
    src = src_ref[0]
    tgt = tgt_ref[0]

    cross = jnp.dot(src.astype(jnp.bfloat16), tgt.astype(jnp.bfloat16),
                    preferred_element_type=jnp.float32)
    sq0 = jnp.sum(src * src, axis=1, keepdims=True)
    sq1 = jnp.sum(tgt * tgt, axis=0, keepdims=True)
    d_sel = sq0 + sq1 - 2.0 * cross

    d_exact = jnp.zeros_like(d_sel)
    for c in range(3):
        diff = src[:, c:c + 1] - tgt[c:c + 1, :]
        d_exact = d_exact + diff * diff

    vals = d_sel
    tops = []
    exacts = []
    for k in range(K):
        m = jnp.min(vals, axis=1, keepdims=True)
        mask = vals == m
        if k > 0:
            tops.append(m)
            exacts.append(jnp.sum(jnp.where(mask, d_exact, 0.0),
                                  axis=1, keepdims=True))
        vals = jnp.where(mask, INF, vals)

    w = [jnp.exp(t * -1000.0) for t in tops]
    s = w[0] + w[1] + w[2] + w[3] + 1e-5
    num = (w[0] * exacts[0] + w[1] * exacts[1]
           + w[2] * exacts[2] + w[3] * exacts[3])
    rows_ref[0] = 0.5 * num / s

    @pl.when(i == 0)
    def _init():
        colv_ref[...] = jnp.full_like(colv_ref, INF)
        colx_ref[...] = jnp.zeros_like(colx_ref)

    bvals = d_sel
    bv = []
    bx = []
    for k in range(K):
        m = jnp.min(bvals, axis=0, keepdims=True)
        mask = bvals == m
        bv.append(m)
        bx.append(jnp.sum(jnp.where(mask, d_exact, 0.0), axis=0, keepdims=True))
        bvals = jnp.where(mask, INF, bvals)

    cand_v = jnp.concatenate([colv_ref[:K]] + bv, axis=0)
    cand_x = jnp.concatenate([colx_ref[:K]] + bx, axis=0)
    cv = []
    cx = []
    for k in range(K):
        m = jnp.min(cand_v, axis=0, keepdims=True)
        mask = cand_v == m
        cv.append(m)
        cx.append(jnp.sum(jnp.where(mask, cand_x, 0.0), axis=0, keepdims=True))
        cand_v = jnp.where(mask, INF, cand_v)
    colv_ref[:K] = jnp.concatenate(cv, axis=0)
    colx_ref[:K] = jnp.concatenate(cx, axis=0)

    @pl.when(i == nb - 1)
    def _final():
        wc = [jnp.exp(cv[k] * -1000.0) for k in range(1, K)]
        sc = wc[0] + wc[1] + wc[2] + wc[3] + 1e-5
        numc = (wc[0] * cx[1] + wc[1] * cx[2]
                + wc[2] * cx[3] + wc[3] * cx[4])
        cols_ref[0] = 0.5 * numc / sc


def kernel(p0, p1):
    B, N, _ = p0.shape
    M = p1.shape[1]
    src_p = jnp.pad(p0, ((0, 0), (0, 0), (0, 5)))
    tgt_t = jnp.pad(p1, ((0, 0), (0, 0), (0, 5))).transpose(0, 2, 1)
    rows, cols = pl.pallas_call(
        _sym_kernel,
        grid=(B, N // BN),
        in_specs=[
            pl.BlockSpec((1, BN, 8), lambda b, i: (b, i, 0)),
            pl.BlockSpec((1, 8, M), lambda b, i: (b, 0, 0)),
        ],
        out_specs=[
            pl.BlockSpec((1, BN, 1), lambda b, i: (b, i, 0)),
            pl.BlockSpec((1, 1, M), lambda b, i: (b, 0, 0)),
        ],
        out_shape=[
            jax.ShapeDtypeStruct((B, N, 1), jnp.float32),
            jax.ShapeDtypeStruct((B, 1, M), jnp.float32),
        ],
        scratch_shapes=[
            pltpu.VMEM((K, M), jnp.float32),
            pltpu.VMEM((K, M), jnp.float32),
        ],
    )(src_p, tgt_t)
    return (jnp.sum(rows) + jnp.sum(cols)) / B


# speedup vs baseline: 1.1793x; 1.0201x over previous
import jax
import jax.numpy as jnp
from jax.experimental import pallas as pl
from jax.experimental.pallas import tpu as pltpu

K = 5
BN = 1024
INF = float("inf")


def _sym_kernel(src_ref, tgt_ref, rows_ref, cols_ref, colv_ref, colx_ref):
    i = pl.program_id(1)
    nb = pl.num_programs(1)
    src = src_ref[0]
    tgt = tgt_ref[0]

    cross = jnp.dot(src.astype(jnp.bfloat16), tgt.astype(jnp.bfloat16),
                    preferred_element_type=jnp.float32)
    sq0 = jnp.sum(src * src, axis=1, keepdims=True)
    sq1 = jnp.sum(tgt * tgt, axis=0, keepdims=True)
    ssum = sq0 + sq1
    d_sel = ssum - 2.0 * cross

    cross_e = src[:, 0:1] * tgt[0:1, :]
    for c in range(1, 3):
        cross_e = cross_e + src[:, c:c + 1] * tgt[c:c + 1, :]
    d_exact = ssum - 2.0 * cross_e

    vals = d_sel
    tops = []
    exacts = []
    for k in range(K):
        m = jnp.min(vals, axis=1, keepdims=True)
        mask = vals == m
        if k > 0:
            tops.append(m)
            exacts.append(jnp.sum(jnp.where(mask, d_exact, 0.0),
                                  axis=1, keepdims=True))
        if k < K - 1:
            vals = jnp.where(mask, INF, vals)

    w = [jnp.exp(t * -1000.0) for t in tops]
    s = w[0] + w[1] + w[2] + w[3] + 1e-5
    num = (w[0] * exacts[0] + w[1] * exacts[1]
           + w[2] * exacts[2] + w[3] * exacts[3])
    rows_ref[0] = 0.5 * num / s

    @pl.when(i == 0)
    def _init():
        colv_ref[...] = jnp.full_like(colv_ref, INF)
        colx_ref[...] = jnp.zeros_like(colx_ref)

    bvals = d_sel
    bv = []
    bx = []
    for k in range(K):
        m = jnp.min(bvals, axis=0, keepdims=True)
        mask = bvals == m
        bv.append(m)
        bx.append(jnp.sum(jnp.where(mask, d_exact, 0.0), axis=0, keepdims=True))
        if k < K - 1:
            bvals = jnp.where(mask, INF, bvals)

    cand_v = jnp.concatenate([colv_ref[:K]] + bv, axis=0)
    cand_x = jnp.concatenate([colx_ref[:K]] + bx, axis=0)
    cv = []
    cx = []
    for k in range(K):
        m = jnp.min(cand_v, axis=0, keepdims=True)
        mask = cand_v == m
        cv.append(m)
        cx.append(jnp.sum(jnp.where(mask, cand_x, 0.0), axis=0, keepdims=True))
        if k < K - 1:
            cand_v = jnp.where(mask, INF, cand_v)
    colv_ref[:K] = jnp.concatenate(cv, axis=0)
    colx_ref[:K] = jnp.concatenate(cx, axis=0)

    @pl.when(i == nb - 1)
    def _final():
        wc = [jnp.exp(cv[k] * -1000.0) for k in range(1, K)]
        sc = wc[0] + wc[1] + wc[2] + wc[3] + 1e-5
        numc = (wc[0] * cx[1] + wc[1] * cx[2]
                + wc[2] * cx[3] + wc[3] * cx[4])
        cols_ref[0] = 0.5 * numc / sc


def kernel(p0, p1):
    B, N, _ = p0.shape
    M = p1.shape[1]
    src_p = jnp.pad(p0, ((0, 0), (0, 0), (0, 5)))
    tgt_t = jnp.pad(p1, ((0, 0), (0, 0), (0, 5))).transpose(0, 2, 1)
    rows, cols = pl.pallas_call(
        _sym_kernel,
        grid=(B, N // BN),
        in_specs=[
            pl.BlockSpec((1, BN, 8), lambda b, i: (b, i, 0)),
            pl.BlockSpec((1, 8, M), lambda b, i: (b, 0, 0)),
        ],
        out_specs=[
            pl.BlockSpec((1, BN, 1), lambda b, i: (b, i, 0)),
            pl.BlockSpec((1, 1, M), lambda b, i: (b, 0, 0)),
        ],
        out_shape=[
            jax.ShapeDtypeStruct((B, N, 1), jnp.float32),
            jax.ShapeDtypeStruct((B, 1, M), jnp.float32),
        ],
        scratch_shapes=[
            pltpu.VMEM((K, M), jnp.float32),
            pltpu.VMEM((K, M), jnp.float32),
        ],
    )(src_p, tgt_t)
    return (jnp.sum(rows) + jnp.sum(cols)) / B


# speedup vs baseline: 1.2057x; 1.0224x over previous
import jax
import jax.numpy as jnp
from jax.experimental import pallas as pl
from jax.experimental.pallas import tpu as pltpu

K = 5
BN = 1024
INF = float("inf")


def _sym_kernel(src_ref, tgt_ref, rows_ref, cols_ref, colv_ref, colx_ref):
    i = pl.program_id(1)
    nb = pl.num_programs(1)
    src = jnp.transpose(src_ref[0])
    tgt = tgt_ref[0]

    cross = jnp.dot(src.astype(jnp.bfloat16), tgt.astype(jnp.bfloat16),
                    preferred_element_type=jnp.float32)
    sq0 = jnp.sum(src * src, axis=1, keepdims=True)
    sq1 = jnp.sum(tgt * tgt, axis=0, keepdims=True)
    ssum = sq0 + sq1
    d_sel = ssum - 2.0 * cross

    cross_e = src[:, 0:1] * tgt[0:1, :]
    for c in range(1, 3):
        cross_e = cross_e + src[:, c:c + 1] * tgt[c:c + 1, :]
    d_exact = ssum - 2.0 * cross_e

    vals = d_sel
    tops = []
    exacts = []
    for k in range(K):
        m = jnp.min(vals, axis=1, keepdims=True)
        mask = vals == m
        if k > 0:
            tops.append(m)
            exacts.append(jnp.sum(jnp.where(mask, d_exact, 0.0),
                                  axis=1, keepdims=True))
        if k < K - 1:
            vals = jnp.where(mask, INF, vals)

    w = [jnp.exp(t * -1000.0) for t in tops]
    s = w[0] + w[1] + w[2] + w[3] + 1e-5
    num = (w[0] * exacts[0] + w[1] * exacts[1]
           + w[2] * exacts[2] + w[3] * exacts[3])
    rows_ref[0] = jnp.transpose(0.5 * num / s)

    @pl.when(i == 0)
    def _init():
        colv_ref[...] = jnp.full_like(colv_ref, INF)
        colx_ref[...] = jnp.zeros_like(colx_ref)

    bvals = d_sel
    bv = []
    bx = []
    for k in range(K):
        m = jnp.min(bvals, axis=0, keepdims=True)
        mask = bvals == m
        bv.append(m)
        bx.append(jnp.sum(jnp.where(mask, d_exact, 0.0), axis=0, keepdims=True))
        if k < K - 1:
            bvals = jnp.where(mask, INF, bvals)

    cand_v = jnp.concatenate([colv_ref[:K]] + bv, axis=0)
    cand_x = jnp.concatenate([colx_ref[:K]] + bx, axis=0)
    cv = []
    cx = []
    for k in range(K):
        m = jnp.min(cand_v, axis=0, keepdims=True)
        mask = cand_v == m
        cv.append(m)
        cx.append(jnp.sum(jnp.where(mask, cand_x, 0.0), axis=0, keepdims=True))
        if k < K - 1:
            cand_v = jnp.where(mask, INF, cand_v)
    colv_ref[:K] = jnp.concatenate(cv, axis=0)
    colx_ref[:K] = jnp.concatenate(cx, axis=0)

    @pl.when(i == nb - 1)
    def _final():
        wc = [jnp.exp(cv[k] * -1000.0) for k in range(1, K)]
        sc = wc[0] + wc[1] + wc[2] + wc[3] + 1e-5
        numc = (wc[0] * cx[1] + wc[1] * cx[2]
                + wc[2] * cx[3] + wc[3] * cx[4])
        cols_ref[0] = 0.5 * numc / sc


def kernel(p0, p1):
    B, N, _ = p0.shape
    M = p1.shape[1]
    src_t = jnp.pad(p0, ((0, 0), (0, 0), (0, 5))).transpose(0, 2, 1)
    tgt_t = jnp.pad(p1, ((0, 0), (0, 0), (0, 5))).transpose(0, 2, 1)
    rows, cols = pl.pallas_call(
        _sym_kernel,
        grid=(B, N // BN),
        in_specs=[
            pl.BlockSpec((1, 8, BN), lambda b, i: (b, 0, i)),
            pl.BlockSpec((1, 8, M), lambda b, i: (b, 0, 0)),
        ],
        out_specs=[
            pl.BlockSpec((1, 1, BN), lambda b, i: (b, 0, i)),
            pl.BlockSpec((1, 1, M), lambda b, i: (b, 0, 0)),
        ],
        out_shape=[
            jax.ShapeDtypeStruct((B, 1, N), jnp.float32),
            jax.ShapeDtypeStruct((B, 1, M), jnp.float32),
        ],
        scratch_shapes=[
            pltpu.VMEM((K, M), jnp.float32),
            pltpu.VMEM((K, M), jnp.float32),
        ],
        compiler_params=pltpu.CompilerParams(
            vmem_limit_bytes=100 * 1024 * 1024),
    )(src_t, tgt_t)
    return (jnp.sum(rows) + jnp.sum(cols)) / B
